# baseline (speedup 1.0000x reference)
import jax
import jax.numpy as jnp
from jax.experimental import pallas as pl
from jax.experimental.pallas import tpu as pltpu

B, C, Hh, Ww = 8, 384, 32, 32
D = 64
K = 1024
HP = Hh + 2
PIX = HP * HP
LP0 = 48
LEXT = 1280
HW = Hh * Ww
SHIFTS = [(dh - 1) * HP + (dw - 1) for dh in range(3) for dw in range(3)]
f32 = jnp.float32
bf16 = jnp.bfloat16


def _pad_in(src_ref, dst_ref, rows, cast):
    dst_ref[...] = jnp.zeros((rows, LEXT), dst_ref.dtype)
    for i in range(Hh):
        blk = src_ref[:, i * Ww:(i + 1) * Ww]
        dst_ref[:, LP0 + (i + 1) * HP + 1:LP0 + (i + 1) * HP + 1 + Ww] = (
            blk.astype(bf16) if cast else blk)


def _conv_outshift(x_ref, b0, nin, wt_ref, b_row, maskl, relu):
    xb = x_ref[b0 * nin:(b0 + 1) * nin, :]
    acc = None
    for k, s in enumerate(SHIFTS):
        p = jax.lax.dot_general(wt_ref[k], xb, (((1,), (0,)), ((), ())),
                                preferred_element_type=f32)
        sh = p[:, LP0 + s:LP0 + s + PIX]
        acc = sh if acc is None else acc + sh
    acc = acc + b_row
    if relu:
        acc = jnp.maximum(acc, 0.0)
    return acc * maskl


def _enc_kernel(h_ref, wq1_ref, bq1_ref, wq2_ref, bq2_ref, cb_ref, cbt_ref,
                maskl_ref,
                zq_out_ref, zqb_out_ref, idx_ref, loss_ref,
                hpad_ref, z1_ref, zeb_ref, zef_ref):
    maskl = maskl_ref[0:1, LP0:LP0 + PIX]
    _pad_in(h_ref, hpad_ref, B * C, True)

    z1_ref[...] = jnp.zeros((B * D, LEXT), bf16)
    for b in range(B):
        z1 = _conv_outshift(hpad_ref, b, C, wq1_ref, bq1_ref[:, 0:1], maskl, True)
        z1_ref[b * D:(b + 1) * D, LP0:LP0 + PIX] = z1.astype(bf16)

    zeb_ref[...] = jnp.zeros((B * D, LEXT), bf16)
    zef_ref[...] = jnp.zeros((B * D, LEXT), f32)
    for b in range(B):
        ze = _conv_outshift(z1_ref, b, D, wq2_ref, bq2_ref[:, 0:1], maskl, False)
        zef_ref[b * D:(b + 1) * D, LP0:LP0 + PIX] = ze
        zeb_ref[b * D:(b + 1) * D, LP0:LP0 + PIX] = ze.astype(bf16)

    zqb_out_ref[...] = jnp.zeros((B * D, LEXT), bf16)
    cb = cb_ref[...]
    cnorm = jnp.sum(cb * cb, axis=1, keepdims=True)
    cb_b = cb.astype(bf16)
    acc_loss = jnp.zeros((1, 1), f32)
    for b in range(B):
        zeb = zeb_ref[b * D:(b + 1) * D, LP0:LP0 + PIX]
        zef = zef_ref[b * D:(b + 1) * D, LP0:LP0 + PIX]
        m = jax.lax.dot_general(cb_b, zeb, (((1,), (0,)), ((), ())),
                                preferred_element_type=f32)
        znorm = jnp.sum(zef * zef, axis=0, keepdims=True)
        dist = (znorm - 2.0 * m) + cnorm
        minv = jnp.min(dist, axis=0, keepdims=True)
        iot = jax.lax.broadcasted_iota(jnp.int32, (K, PIX), 0)
        idx = jnp.min(jnp.where(dist == minv, iot, K), axis=0, keepdims=True)
        idx_ref[b:b + 1, LP0:LP0 + PIX] = idx
        onehot = (iot == idx).astype(f32)
        zq = jax.lax.dot_general(cbt_ref[...], onehot, (((1,), (0,)), ((), ())),
                                 preferred_element_type=f32,
                                 precision=jax.lax.Precision.HIGHEST)
        zq = zq * maskl
        diff = zef - zq
        acc_loss = acc_loss + jnp.sum(diff * diff).reshape(1, 1)
        zqb_out_ref[b * D:(b + 1) * D, LP0:LP0 + PIX] = zq.astype(bf16)
        for i in range(Hh):
            zq_out_ref[b * D:(b + 1) * D, i * Ww:(i + 1) * Ww] = (
                zq[:, (i + 1) * HP + 1:(i + 1) * HP + 1 + Ww])
    loss_ref[...] = acc_loss * (1.0 / (B * HW * D))


def _dec_kernel(zqb_ref, wr1_ref, br1_ref, wr2_ref, br2_ref, wsk_ref, bsk_ref,
                maskl_ref, hhat_ref, r1_ref):
    maskl = maskl_ref[0:1, LP0:LP0 + PIX]
    r1_ref[...] = jnp.zeros((B * C, LEXT), bf16)
    for b in range(B):
        r1 = _conv_outshift(zqb_ref, b, D, wr1_ref, br1_ref[:, 0:1], maskl, True)
        r1_ref[b * C:(b + 1) * C, LP0:LP0 + PIX] = r1.astype(bf16)
    for b in range(B):
        xb = r1_ref[b * C:(b + 1) * C, :]
        acc = None
        for k, s in enumerate(SHIFTS):
            x = xb[:, LP0 + s:LP0 + s + PIX]
            p = jax.lax.dot_general(wr2_ref[k], x, (((1,), (0,)), ((), ())),
                                    preferred_element_type=f32)
            acc = p if acc is None else acc + p
        ysk = jax.lax.dot_general(wsk_ref[...],
                                  zqb_ref[b * D:(b + 1) * D, LP0:LP0 + PIX],
                                  (((1,), (0,)), ((), ())),
                                  preferred_element_type=f32)
        res = (acc + br2_ref[:, 0:1]) + (ysk + bsk_ref[:, 0:1])
        for i in range(Hh):
            hhat_ref[b * C:(b + 1) * C, i * Ww:(i + 1) * Ww] = (
                res[:, (i + 1) * HP + 1:(i + 1) * HP + 1 + Ww])


def kernel(h, Wq1, bq1, Wq2, bq2, codebook, Wr1, br1, Wr2, br2, Wskip, bskip):
    h2 = h.reshape(B * C, HW)
    wq1 = jnp.transpose(Wq1, (2, 3, 0, 1)).reshape(9, D, C).astype(bf16)
    wq2 = jnp.transpose(Wq2, (2, 3, 0, 1)).reshape(9, D, D).astype(bf16)
    wr1 = jnp.transpose(Wr1, (2, 3, 0, 1)).reshape(9, C, D).astype(bf16)
    wr2 = jnp.transpose(Wr2, (2, 3, 0, 1)).reshape(9, C, C).astype(bf16)
    wsk = jnp.transpose(Wskip, (2, 3, 0, 1)).reshape(C, D).astype(bf16)
    cbt = codebook.T

    l = jnp.arange(LEXT) - LP0
    li = l // HP
    lj = l % HP
    valid = (l >= 0) & (l < PIX) & (li >= 1) & (li <= Hh) & (lj >= 1) & (lj <= Ww)
    maskl = valid.astype(f32)[None, :]

    zq_rows, zqb, idx_pad, loss = pl.pallas_call(
        _enc_kernel,
        out_shape=(
            jax.ShapeDtypeStruct((B * D, HW), f32),
            jax.ShapeDtypeStruct((B * D, LEXT), bf16),
            jax.ShapeDtypeStruct((B, LEXT), jnp.int32),
            jax.ShapeDtypeStruct((1, 1), f32),
        ),
        scratch_shapes=[
            pltpu.VMEM((B * C, LEXT), bf16),
            pltpu.VMEM((B * D, LEXT), bf16),
            pltpu.VMEM((B * D, LEXT), bf16),
            pltpu.VMEM((B * D, LEXT), f32),
        ],
        compiler_params=pltpu.CompilerParams(
            vmem_limit_bytes=100 * 1024 * 1024,
        ),
    )(h2, wq1, bq1.reshape(D, 1), wq2, bq2.reshape(D, 1), codebook, cbt, maskl)

    hhat = pl.pallas_call(
        _dec_kernel,
        out_shape=jax.ShapeDtypeStruct((B * C, HW), f32),
        scratch_shapes=[pltpu.VMEM((B * C, LEXT), bf16)],
        compiler_params=pltpu.CompilerParams(
            vmem_limit_bytes=100 * 1024 * 1024,
        ),
    )(zqb, wr1, br1.reshape(C, 1), wr2, br2.reshape(C, 1), wsk,
      bskip.reshape(C, 1), maskl)

    z_q_st = zq_rows.reshape(B, D, Hh, Ww)
    h_hat = hhat.reshape(B, C, Hh, Ww)
    idx = idx_pad[:, LP0:LP0 + PIX].reshape(B, HP, HP)[:, 1:1 + Hh, 1:1 + Ww]
    return (z_q_st, h_hat, loss.reshape(()), idx)


# speedup vs baseline: 1.1665x; 1.1665x over previous
import jax
import jax.numpy as jnp
from jax.experimental import pallas as pl
from jax.experimental.pallas import tpu as pltpu

B, C, Hh, Ww = 8, 384, 32, 32
D = 64
K = 1024
HP = Hh + 2
ROWS = B * HP * HP
PAD0 = 48
EXT = 9344
VQC = 8
VQR = EXT // VQC
CC = 4
CR = ROWS // CC
SHIFTS = [(dh - 1) * HP + (dw - 1) for dh in range(3) for dw in range(3)]
f32 = jnp.float32
bf16 = jnp.bfloat16


def _conv9_chunked(x_ref, w_ref, b_row, out_ref, relu, mask_ref, nout):
    for c in range(CC):
        base = PAD0 + c * CR
        acc = None
        for k, s in enumerate(SHIFTS):
            x = x_ref[base + s:base + s + CR, :]
            if x.dtype != bf16:
                x = x.astype(bf16)
            p = jax.lax.dot_general(x, w_ref[k], (((1,), (0,)), ((), ())),
                                    preferred_element_type=f32)
            acc = p if acc is None else acc + p
        acc = acc + b_row
        if relu:
            acc = jnp.maximum(acc, 0.0)
        out = acc * mask_ref[base:base + CR, :]
        out_ref[base:base + CR, :] = out.astype(out_ref.dtype)


def _enc_kernel(h_ref, wq1_ref, bq1_ref, wq2_ref, bq2_ref, cb_ref, mask_ref,
                zq_ref, idx_ref, loss_ref, z1_ref, ze_ref):
    z1_ref[...] = jnp.zeros((EXT, D), bf16)
    ze_ref[...] = jnp.zeros((EXT, D), f32)
    _conv9_chunked(h_ref, wq1_ref, bq1_ref[0:1, :], z1_ref, True, mask_ref, D)
    _conv9_chunked(z1_ref, wq2_ref, bq2_ref[0:1, :], ze_ref, False, mask_ref, D)

    cb = cb_ref[...]
    cb_b = cb.astype(bf16)
    cnorm = jnp.sum(cb * cb, axis=1, keepdims=True).reshape(1, K)
    acc_loss = jnp.zeros((1, 1), f32)
    for c in range(VQC):
        z = ze_ref[c * VQR:(c + 1) * VQR, :]
        m = jax.lax.dot_general(z.astype(bf16), cb_b, (((1,), (1,)), ((), ())),
                                preferred_element_type=f32)
        znorm = jnp.sum(z * z, axis=1, keepdims=True)
        dist = (znorm - 2.0 * m) + cnorm
        minv = jnp.min(dist, axis=1, keepdims=True)
        iot = jax.lax.broadcasted_iota(jnp.int32, (VQR, K), 1)
        idx = jnp.min(jnp.where(dist == minv, iot, K), axis=1, keepdims=True)
        idx_ref[c * VQR:(c + 1) * VQR, :] = idx
        onehot = (iot == idx).astype(f32)
        zq = jax.lax.dot_general(onehot, cb, (((1,), (0,)), ((), ())),
                                 preferred_element_type=f32,
                                 precision=jax.lax.Precision.HIGHEST)
        zq = zq * mask_ref[c * VQR:(c + 1) * VQR, :]
        zq_ref[c * VQR:(c + 1) * VQR, :] = zq
        diff = z - zq
        acc_loss = acc_loss + jnp.sum(diff * diff).reshape(1, 1)
    loss_ref[...] = acc_loss * (1.0 / (B * Hh * Ww * D))


def _dec_kernel(zq_ref, wr1_ref, br1_ref, wr2_ref, br2_ref, wsk_ref, bsk_ref,
                mask_ref, hhat_ref, r1_ref):
    r1_ref[...] = jnp.zeros((EXT, C), bf16)
    _conv9_chunked(zq_ref, wr1_ref, br1_ref[0:1, :], r1_ref, True, mask_ref, C)
    for c in range(CC):
        base = PAD0 + c * CR
        acc = None
        for k, s in enumerate(SHIFTS):
            x = r1_ref[base + s:base + s + CR, :]
            p = jax.lax.dot_general(x, wr2_ref[k], (((1,), (0,)), ((), ())),
                                    preferred_element_type=f32)
            acc = p if acc is None else acc + p
        ysk = jax.lax.dot_general(zq_ref[base:base + CR, :].astype(bf16), wsk_ref[...],
                                  (((1,), (0,)), ((), ())),
                                  preferred_element_type=f32)
        hhat_ref[c * CR:(c + 1) * CR, :] = (acc + br2_ref[0:1, :]) + (ysk + bsk_ref[0:1, :])


def kernel(h, Wq1, bq1, Wq2, bq2, codebook, Wr1, br1, Wr2, br2, Wskip, bskip):
    hp = jnp.pad(jnp.transpose(h, (0, 2, 3, 1)), ((0, 0), (1, 1), (1, 1), (0, 0)))
    hflat = hp.reshape(ROWS, C)
    h_ext = jnp.pad(hflat, ((PAD0, EXT - PAD0 - ROWS), (0, 0))).astype(bf16)

    wq1 = jnp.transpose(Wq1, (2, 3, 1, 0)).reshape(9, C, D).astype(bf16)
    wq2 = jnp.transpose(Wq2, (2, 3, 1, 0)).reshape(9, D, D).astype(bf16)
    wr1 = jnp.transpose(Wr1, (2, 3, 1, 0)).reshape(9, D, C).astype(bf16)
    wr2 = jnp.transpose(Wr2, (2, 3, 1, 0)).reshape(9, C, C).astype(bf16)
    wsk = jnp.transpose(Wskip, (2, 3, 1, 0)).reshape(D, C).astype(bf16)

    r = jnp.arange(EXT) - PAD0
    j = r % (HP * HP) % HP
    i = r % (HP * HP) // HP
    valid = (r >= 0) & (r < ROWS) & (i >= 1) & (i <= Hh) & (j >= 1) & (j <= Ww)
    mask = valid.astype(f32)[:, None]

    zq_ext, idx_ext, loss = pl.pallas_call(
        _enc_kernel,
        out_shape=(
            jax.ShapeDtypeStruct((EXT, D), f32),
            jax.ShapeDtypeStruct((EXT, 1), jnp.int32),
            jax.ShapeDtypeStruct((1, 1), f32),
        ),
        scratch_shapes=[
            pltpu.VMEM((EXT, D), bf16),
            pltpu.VMEM((EXT, D), f32),
        ],
        compiler_params=pltpu.CompilerParams(
            vmem_limit_bytes=100 * 1024 * 1024,
        ),
    )(h_ext, wq1, bq1.reshape(1, D), wq2, bq2.reshape(1, D), codebook, mask)

    hhat_rows = pl.pallas_call(
        _dec_kernel,
        out_shape=jax.ShapeDtypeStruct((ROWS, C), f32),
        scratch_shapes=[pltpu.VMEM((EXT, C), bf16)],
        compiler_params=pltpu.CompilerParams(
            vmem_limit_bytes=100 * 1024 * 1024,
        ),
    )(zq_ext, wr1, br1.reshape(1, C), wr2, br2.reshape(1, C), wsk,
      bskip.reshape(1, C), mask)

    zq = zq_ext[PAD0:PAD0 + ROWS].reshape(B, HP, HP, D)[:, 1:1 + Hh, 1:1 + Ww, :]
    z_q_st = jnp.transpose(zq, (0, 3, 1, 2))
    hh = hhat_rows.reshape(B, HP, HP, C)[:, 1:1 + Hh, 1:1 + Ww, :]
    h_hat = jnp.transpose(hh, (0, 3, 1, 2))
    indices = idx_ext[PAD0:PAD0 + ROWS, 0].reshape(B, HP, HP)[:, 1:1 + Hh, 1:1 + Ww]
    return (z_q_st, h_hat, loss.reshape(()), indices)


# speedup vs baseline: 1.2652x; 1.0847x over previous
import jax
import jax.numpy as jnp
from jax.experimental import pallas as pl
from jax.experimental.pallas import tpu as pltpu

B, C, Hh, Ww = 8, 384, 32, 32
D = 64
K = 1024
HP = Hh + 2
ROWS = B * HP * HP
PAD0 = 48
EXT = 9344
VQC = 8
VQR = EXT // VQC
CC = 4
CR = ROWS // CC
SHIFTS = [(dh - 1) * HP + (dw - 1) for dh in range(3) for dw in range(3)]
f32 = jnp.float32
bf16 = jnp.bfloat16


def _conv9_chunked(x_ref, w_ref, b_row, out_ref, relu, mask_ref, nout):
    for c in range(CC):
        base = PAD0 + c * CR
        acc = None
        for k, s in enumerate(SHIFTS):
            x = x_ref[base + s:base + s + CR, :]
            if x.dtype != bf16:
                x = x.astype(bf16)
            p = jax.lax.dot_general(x, w_ref[k], (((1,), (0,)), ((), ())),
                                    preferred_element_type=f32)
            acc = p if acc is None else acc + p
        acc = acc + b_row
        if relu:
            acc = jnp.maximum(acc, 0.0)
        out = acc * mask_ref[base:base + CR, :]
        out_ref[base:base + CR, :] = out.astype(out_ref.dtype)


def _enc_kernel(h_ref, wq1_ref, bq1_ref, wq2_ref, bq2_ref, cb_ref, mask_ref,
                zq_ref, idx_ref, loss_ref, z1_ref, ze_ref):
    z1_ref[...] = jnp.zeros((EXT, D), bf16)
    ze_ref[...] = jnp.zeros((EXT, D), f32)
    _conv9_chunked(h_ref, wq1_ref, bq1_ref[0:1, :], z1_ref, True, mask_ref, D)
    _conv9_chunked(z1_ref, wq2_ref, bq2_ref[0:1, :], ze_ref, False, mask_ref, D)

    cb = cb_ref[...]
    cb_b = cb.astype(bf16)
    cb_lo = (cb - cb_b.astype(f32)).astype(bf16)
    cnorm = jnp.sum(cb * cb, axis=1, keepdims=True).reshape(1, K)
    acc_loss = jnp.zeros((1, 1), f32)
    for c in range(VQC):
        z = ze_ref[c * VQR:(c + 1) * VQR, :]
        m = jax.lax.dot_general(z.astype(bf16), cb_b, (((1,), (1,)), ((), ())),
                                preferred_element_type=f32)
        znorm = jnp.sum(z * z, axis=1, keepdims=True)
        dist = (znorm - 2.0 * m) + cnorm
        minv = jnp.min(dist, axis=1, keepdims=True)
        iot = jax.lax.broadcasted_iota(jnp.int32, (VQR, K), 1)
        idx = jnp.min(jnp.where(dist == minv, iot, K), axis=1, keepdims=True)
        idx_ref[c * VQR:(c + 1) * VQR, :] = idx
        onehot = (iot == idx).astype(bf16)
        zq = (jax.lax.dot_general(onehot, cb_b, (((1,), (0,)), ((), ())),
                                  preferred_element_type=f32)
              + jax.lax.dot_general(onehot, cb_lo, (((1,), (0,)), ((), ())),
                                    preferred_element_type=f32))
        zq = zq * mask_ref[c * VQR:(c + 1) * VQR, :]
        zq_ref[c * VQR:(c + 1) * VQR, :] = zq
        diff = z - zq
        acc_loss = acc_loss + jnp.sum(diff * diff).reshape(1, 1)
    loss_ref[...] = acc_loss * (1.0 / (B * Hh * Ww * D))


def _dec_kernel(zq_ref, wr1_ref, br1_ref, wr2_ref, br2_ref, wsk_ref, bsk_ref,
                mask_ref, hhat_ref, r1_ref):
    r1_ref[...] = jnp.zeros((EXT, C), bf16)
    _conv9_chunked(zq_ref, wr1_ref, br1_ref[0:1, :], r1_ref, True, mask_ref, C)
    for c in range(CC):
        base = PAD0 + c * CR
        acc = None
        for k, s in enumerate(SHIFTS):
            x = r1_ref[base + s:base + s + CR, :]
            p = jax.lax.dot_general(x, wr2_ref[k], (((1,), (0,)), ((), ())),
                                    preferred_element_type=f32)
            acc = p if acc is None else acc + p
        ysk = jax.lax.dot_general(zq_ref[base:base + CR, :].astype(bf16), wsk_ref[...],
                                  (((1,), (0,)), ((), ())),
                                  preferred_element_type=f32)
        hhat_ref[c * CR:(c + 1) * CR, :] = (acc + br2_ref[0:1, :]) + (ysk + bsk_ref[0:1, :])


def kernel(h, Wq1, bq1, Wq2, bq2, codebook, Wr1, br1, Wr2, br2, Wskip, bskip):
    hp = jnp.pad(jnp.transpose(h, (0, 2, 3, 1)), ((0, 0), (1, 1), (1, 1), (0, 0)))
    hflat = hp.reshape(ROWS, C)
    h_ext = jnp.pad(hflat, ((PAD0, EXT - PAD0 - ROWS), (0, 0))).astype(bf16)

    wq1 = jnp.transpose(Wq1, (2, 3, 1, 0)).reshape(9, C, D).astype(bf16)
    wq2 = jnp.transpose(Wq2, (2, 3, 1, 0)).reshape(9, D, D).astype(bf16)
    wr1 = jnp.transpose(Wr1, (2, 3, 1, 0)).reshape(9, D, C).astype(bf16)
    wr2 = jnp.transpose(Wr2, (2, 3, 1, 0)).reshape(9, C, C).astype(bf16)
    wsk = jnp.transpose(Wskip, (2, 3, 1, 0)).reshape(D, C).astype(bf16)

    r = jnp.arange(EXT) - PAD0
    j = r % (HP * HP) % HP
    i = r % (HP * HP) // HP
    valid = (r >= 0) & (r < ROWS) & (i >= 1) & (i <= Hh) & (j >= 1) & (j <= Ww)
    mask = valid.astype(f32)[:, None]

    zq_ext, idx_ext, loss = pl.pallas_call(
        _enc_kernel,
        out_shape=(
            jax.ShapeDtypeStruct((EXT, D), f32),
            jax.ShapeDtypeStruct((EXT, 1), jnp.int32),
            jax.ShapeDtypeStruct((1, 1), f32),
        ),
        scratch_shapes=[
            pltpu.VMEM((EXT, D), bf16),
            pltpu.VMEM((EXT, D), f32),
        ],
        compiler_params=pltpu.CompilerParams(
            vmem_limit_bytes=100 * 1024 * 1024,
        ),
    )(h_ext, wq1, bq1.reshape(1, D), wq2, bq2.reshape(1, D), codebook, mask)

    hhat_rows = pl.pallas_call(
        _dec_kernel,
        out_shape=jax.ShapeDtypeStruct((ROWS, C), f32),
        scratch_shapes=[pltpu.VMEM((EXT, C), bf16)],
        compiler_params=pltpu.CompilerParams(
            vmem_limit_bytes=100 * 1024 * 1024,
        ),
    )(zq_ext, wr1, br1.reshape(1, C), wr2, br2.reshape(1, C), wsk,
      bskip.reshape(1, C), mask)

    zq = zq_ext[PAD0:PAD0 + ROWS].reshape(B, HP, HP, D)[:, 1:1 + Hh, 1:1 + Ww, :]
    z_q_st = jnp.transpose(zq, (0, 3, 1, 2))
    hh = hhat_rows.reshape(B, HP, HP, C)[:, 1:1 + Hh, 1:1 + Ww, :]
    h_hat = jnp.transpose(hh, (0, 3, 1, 2))
    indices = idx_ext[PAD0:PAD0 + ROWS, 0].reshape(B, HP, HP)[:, 1:1 + Hh, 1:1 + Ww]
    return (z_q_st, h_hat, loss.reshape(()), indices)


# speedup vs baseline: 1.4246x; 1.1260x over previous
import jax
import jax.numpy as jnp
from jax.experimental import pallas as pl
from jax.experimental.pallas import tpu as pltpu

B, C, Hh, Ww = 8, 384, 32, 32
D = 64
K = 1024
HP = Hh + 2
ROWS = B * HP * HP
PAD0 = 48
EXT = 9344
VQC = 8
VQR = EXT // VQC
CC = 4
CR = ROWS // CC
SHIFTS = [(dh - 1) * HP + (dw - 1) for dh in range(3) for dw in range(3)]
GROUPS = [(0, 0, 4), (1, 4, 4), (2, 8, 1)]
f32 = jnp.float32
bf16 = jnp.bfloat16


def _conv9(x_ref, w_ref, b_row, mask_ref, relu, nout, base):
    parts = []
    for gi, g0, gn in GROUPS:
        s0 = SHIFTS[g0]
        span = CR + (SHIFTS[g0 + gn - 1] - s0)
        x = x_ref[base + s0:base + s0 + span, :]
        if x.dtype != bf16:
            x = x.astype(bf16)
        y = jax.lax.dot_general(x, w_ref[gi], (((1,), (0,)), ((), ())),
                                preferred_element_type=f32)
        for i in range(gn):
            d = SHIFTS[g0 + i] - s0
            parts.append(y[d:d + CR, i * nout:(i + 1) * nout])
    acc = None
    for p in parts:
        acc = p if acc is None else acc + p
    acc = acc + b_row
    if relu:
        acc = jnp.maximum(acc, 0.0)
    return acc * mask_ref[base:base + CR, :]


def _enc_kernel(h_ref, wq1_ref, bq1_ref, wq2_ref, bq2_ref, cb_ref, mask_ref,
                zq_ref, idx_ref, loss_ref, z1_ref, ze_ref):
    z1_ref[...] = jnp.zeros((EXT, D), bf16)
    ze_ref[...] = jnp.zeros((EXT, D), f32)
    for c in range(CC):
        base = PAD0 + c * CR
        z1 = _conv9(h_ref, wq1_ref, bq1_ref[0:1, :], mask_ref, True, D, base)
        z1_ref[base:base + CR, :] = z1.astype(bf16)
    for c in range(CC):
        base = PAD0 + c * CR
        ze = _conv9(z1_ref, wq2_ref, bq2_ref[0:1, :], mask_ref, False, D, base)
        ze_ref[base:base + CR, :] = ze

    cb = cb_ref[...]
    cb_b = cb.astype(bf16)
    cb_lo = (cb - cb_b.astype(f32)).astype(bf16)
    cnorm = jnp.sum(cb * cb, axis=1, keepdims=True).reshape(1, K)
    acc_loss = jnp.zeros((1, 1), f32)
    for c in range(VQC):
        z = ze_ref[c * VQR:(c + 1) * VQR, :]
        m = jax.lax.dot_general(z.astype(bf16), cb_b, (((1,), (1,)), ((), ())),
                                preferred_element_type=f32)
        znorm = jnp.sum(z * z, axis=1, keepdims=True)
        dist = (znorm - 2.0 * m) + cnorm
        minv = jnp.min(dist, axis=1, keepdims=True)
        iot = jax.lax.broadcasted_iota(jnp.int32, (VQR, K), 1)
        idx = jnp.min(jnp.where(dist == minv, iot, K), axis=1, keepdims=True)
        idx_ref[c * VQR:(c + 1) * VQR, :] = idx
        onehot = (iot == idx).astype(bf16)
        zq = (jax.lax.dot_general(onehot, cb_b, (((1,), (0,)), ((), ())),
                                  preferred_element_type=f32)
              + jax.lax.dot_general(onehot, cb_lo, (((1,), (0,)), ((), ())),
                                    preferred_element_type=f32))
        zq = zq * mask_ref[c * VQR:(c + 1) * VQR, :]
        zq_ref[c * VQR:(c + 1) * VQR, :] = zq
        diff = z - zq
        acc_loss = acc_loss + jnp.sum(diff * diff).reshape(1, 1)
    loss_ref[...] = acc_loss * (1.0 / (B * Hh * Ww * D))


def _dec_kernel(zq_ref, wr1_ref, br1_ref, wr2_ref, br2_ref, wsk_ref, bsk_ref,
                mask_ref, hhat_ref, r1_ref):
    r1_ref[...] = jnp.zeros((EXT, C), bf16)
    for c in range(CC):
        base = PAD0 + c * CR
        r1 = _conv9(zq_ref, wr1_ref, br1_ref[0:1, :], mask_ref, True, C, base)
        r1_ref[base:base + CR, :] = r1.astype(bf16)
    for c in range(CC):
        base = PAD0 + c * CR
        parts = []
        for gi, g0, gn in GROUPS:
            s0 = SHIFTS[g0]
            span = CR + (SHIFTS[g0 + gn - 1] - s0)
            x = r1_ref[base + s0:base + s0 + span, :]
            y = jax.lax.dot_general(x, wr2_ref[gi], (((1,), (0,)), ((), ())),
                                    preferred_element_type=f32)
            for i in range(gn):
                d = SHIFTS[g0 + i] - s0
                parts.append(y[d:d + CR, i * C:(i + 1) * C])
        acc = None
        for p in parts:
            acc = p if acc is None else acc + p
        ysk = jax.lax.dot_general(zq_ref[base:base + CR, :].astype(bf16),
                                  wsk_ref[...], (((1,), (0,)), ((), ())),
                                  preferred_element_type=f32)
        hhat_ref[(base - PAD0):(base - PAD0) + CR, :] = (
            (acc + br2_ref[0:1, :]) + (ysk + bsk_ref[0:1, :]))


def _packw(wt, nout):
    slabs = []
    for gi, g0, gn in GROUPS:
        cat = jnp.concatenate([wt[g0 + i] for i in range(gn)], axis=1)
        if gn < 4:
            cat = jnp.pad(cat, ((0, 0), (0, (4 - gn) * nout)))
        slabs.append(cat)
    return jnp.stack(slabs)


def kernel(h, Wq1, bq1, Wq2, bq2, codebook, Wr1, br1, Wr2, br2, Wskip, bskip):
    hp = jnp.pad(jnp.transpose(h, (0, 2, 3, 1)), ((0, 0), (1, 1), (1, 1), (0, 0)))
    hflat = hp.reshape(ROWS, C)
    h_ext = jnp.pad(hflat, ((PAD0, EXT - PAD0 - ROWS), (0, 0))).astype(bf16)

    wq1 = _packw(jnp.transpose(Wq1, (2, 3, 1, 0)).reshape(9, C, D).astype(bf16), D)
    wq2 = _packw(jnp.transpose(Wq2, (2, 3, 1, 0)).reshape(9, D, D).astype(bf16), D)
    wr1 = _packw(jnp.transpose(Wr1, (2, 3, 1, 0)).reshape(9, D, C).astype(bf16), C)
    wr2 = _packw(jnp.transpose(Wr2, (2, 3, 1, 0)).reshape(9, C, C).astype(bf16), C)
    wsk = jnp.transpose(Wskip, (2, 3, 1, 0)).reshape(D, C).astype(bf16)

    r = jnp.arange(EXT) - PAD0
    j = r % (HP * HP) % HP
    i = r % (HP * HP) // HP
    valid = (r >= 0) & (r < ROWS) & (i >= 1) & (i <= Hh) & (j >= 1) & (j <= Ww)
    mask = valid.astype(f32)[:, None]

    zq_ext, idx_ext, loss = pl.pallas_call(
        _enc_kernel,
        out_shape=(
            jax.ShapeDtypeStruct((EXT, D), f32),
            jax.ShapeDtypeStruct((EXT, 1), jnp.int32),
            jax.ShapeDtypeStruct((1, 1), f32),
        ),
        scratch_shapes=[
            pltpu.VMEM((EXT, D), bf16),
            pltpu.VMEM((EXT, D), f32),
        ],
        compiler_params=pltpu.CompilerParams(
            vmem_limit_bytes=100 * 1024 * 1024,
        ),
    )(h_ext, wq1, bq1.reshape(1, D), wq2, bq2.reshape(1, D), codebook, mask)

    hhat_rows = pl.pallas_call(
        _dec_kernel,
        out_shape=jax.ShapeDtypeStruct((ROWS, C), f32),
        scratch_shapes=[pltpu.VMEM((EXT, C), bf16)],
        compiler_params=pltpu.CompilerParams(
            vmem_limit_bytes=100 * 1024 * 1024,
        ),
    )(zq_ext, wr1, br1.reshape(1, C), wr2, br2.reshape(1, C), wsk,
      bskip.reshape(1, C), mask)

    zq = zq_ext[PAD0:PAD0 + ROWS].reshape(B, HP, HP, D)[:, 1:1 + Hh, 1:1 + Ww, :]
    z_q_st = jnp.transpose(zq, (0, 3, 1, 2))
    hh = hhat_rows.reshape(B, HP, HP, C)[:, 1:1 + Hh, 1:1 + Ww, :]
    h_hat = jnp.transpose(hh, (0, 3, 1, 2))
    indices = idx_ext[PAD0:PAD0 + ROWS, 0].reshape(B, HP, HP)[:, 1:1 + Hh, 1:1 + Ww]
    return (z_q_st, h_hat, loss.reshape(()), indices)
